# Initial kernel scaffold; baseline (speedup 1.0000x reference)
import jax
import jax.numpy as jnp
from jax.experimental import pallas as pl


def kernel(token, token_len, mention_theme, mention_senti, emb_table, W_theme, K_theme, W_senti, K_senti, U_theme, b_theme, U_senti, b_senti):
    raise NotImplementedError("write your pallas kernel here")


# speedup vs baseline: 2.4118x; 2.4118x over previous
import functools

import jax
import jax.numpy as jnp
from jax import lax
from jax.experimental import pallas as pl
from jax.experimental.pallas import tpu as pltpu
from jax.experimental.pallas import tpu_sc as plsc

VOCAB = 100000
EMB = 64
HID = 128
N_THEME = 10
N_SENTI = 3
B = 1024
L = 200
BL = B * L

NC = 2
NSUB = 16
NW = NC * NSUB
TOK_PER_W = BL // NW
CHUNK = 128
N_CHUNKS = TOK_PER_W // CHUNK

BB = 8


def _sc_gather(token_flat, table128):
    mesh = plsc.VectorSubcoreMesh(core_axis_name="c", subcore_axis_name="s")

    @functools.partial(
        pl.kernel,
        out_type=jax.ShapeDtypeStruct((BL, HID), jnp.float32),
        mesh=mesh,
        scratch_types=[
            pltpu.VMEM((CHUNK,), jnp.int32),
            pltpu.VMEM((CHUNK, HID), jnp.float32),
            pltpu.SemaphoreType.DMA,
        ],
    )
    def gather_kernel(tok_hbm, table_hbm, out_hbm, idx_v, rows_v, sem):
        wid = lax.axis_index("s") * NC + lax.axis_index("c")
        base = wid * TOK_PER_W

        def body(c, carry):
            off = base + c * CHUNK
            pltpu.sync_copy(tok_hbm.at[pl.ds(off, CHUNK)], idx_v)
            pltpu.async_copy(table_hbm.at[idx_v], rows_v, sem).wait()
            pltpu.sync_copy(rows_v, out_hbm.at[pl.ds(off, CHUNK)])
            return carry

        lax.fori_loop(0, N_CHUNKS, body, 0)

    return gather_kernel(token_flat, table128)


def _tc_body(len_ref, emb_ref, mt_ref, ms_ref, wt_ref, kt_ref, ws_ref,
             ks_ref, ut_ref, bt_ref, us_ref, bs_ref, out_t_ref, out_s_ref):
    embx = emb_ref[...]
    zt = (jnp.dot(embx, wt_ref[...], preferred_element_type=jnp.float32)
          + jnp.dot(mt_ref[...], kt_ref[...], preferred_element_type=jnp.float32))
    zs = (jnp.dot(embx, ws_ref[...], preferred_element_type=jnp.float32)
          + jnp.dot(ms_ref[...], ks_ref[...], preferred_element_type=jnp.float32))
    ht = jnp.tanh(zt)
    hs = jnp.tanh(zs)

    lens = len_ref[...]
    rows = lax.broadcasted_iota(jnp.int32, (BB, BB * L), 0)
    cols = lax.broadcasted_iota(jnp.int32, (BB, BB * L), 1)
    q = cols // L
    r = cols - q * L
    valid = (q == rows) & (r < lens)
    sel = jnp.where(valid, 1.0, 0.0)

    denom = jnp.maximum(lens.astype(jnp.float32), 1.0)
    pooled_t = jnp.dot(sel, ht, preferred_element_type=jnp.float32) / denom
    pooled_s = jnp.dot(sel, hs, preferred_element_type=jnp.float32) / denom
    out_t_ref[...] = (jnp.dot(pooled_t, ut_ref[...],
                              preferred_element_type=jnp.float32) + bt_ref[...])
    out_s_ref[...] = (jnp.dot(pooled_s, us_ref[...],
                              preferred_element_type=jnp.float32) + bs_ref[...])


def _tc_forward(len2, emb_flat, mt2, ms2, W_theme, K_theme, W_senti, K_senti,
                U_theme, bt2, U_senti, bs2):
    grid = (B // BB,)
    full = lambda shape: pl.BlockSpec(shape, lambda i: (0, 0))
    return pl.pallas_call(
        _tc_body,
        grid=grid,
        in_specs=[
            pl.BlockSpec((BB, 1), lambda i: (i, 0)),
            pl.BlockSpec((BB * L, HID), lambda i: (i, 0)),
            pl.BlockSpec((BB * L, N_THEME), lambda i: (i, 0)),
            pl.BlockSpec((BB * L, N_SENTI), lambda i: (i, 0)),
            full((HID, HID)),
            full((N_THEME, HID)),
            full((HID, HID)),
            full((N_SENTI, HID)),
            full((HID, N_THEME)),
            full((1, N_THEME)),
            full((HID, N_SENTI)),
            full((1, N_SENTI)),
        ],
        out_specs=(
            pl.BlockSpec((BB, N_THEME), lambda i: (i, 0)),
            pl.BlockSpec((BB, N_SENTI), lambda i: (i, 0)),
        ),
        out_shape=(
            jax.ShapeDtypeStruct((B, N_THEME), jnp.float32),
            jax.ShapeDtypeStruct((B, N_SENTI), jnp.float32),
        ),
    )(len2, emb_flat, mt2, ms2, W_theme, K_theme, W_senti, K_senti,
      U_theme, bt2, U_senti, bs2)


def kernel(token, token_len, mention_theme, mention_senti, emb_table,
           W_theme, K_theme, W_senti, K_senti,
           U_theme, b_theme, U_senti, b_senti):
    token_flat = token.reshape(BL)
    table128 = jnp.pad(emb_table, ((0, 0), (0, HID - EMB)))
    wt128 = jnp.pad(W_theme, ((0, HID - EMB), (0, 0)))
    ws128 = jnp.pad(W_senti, ((0, HID - EMB), (0, 0)))
    emb_flat = _sc_gather(token_flat, table128)
    len2 = token_len.reshape(B, 1)
    mt2 = mention_theme.reshape(BL, N_THEME)
    ms2 = mention_senti.reshape(BL, N_SENTI)
    bt2 = b_theme.reshape(1, N_THEME)
    bs2 = b_senti.reshape(1, N_SENTI)
    return _tc_forward(len2, emb_flat, mt2, ms2, wt128, K_theme,
                       ws128, K_senti, U_theme, bt2, U_senti, bs2)
